# initial kernel scaffold (unmeasured)
import jax
import jax.numpy as jnp
from jax import lax
from jax.experimental import pallas as pl
from jax.experimental.pallas import tpu as pltpu


def kernel(
    x,
):
    def body(*refs):
        pass

    out_shape = jax.ShapeDtypeStruct(..., jnp.float32)
    return pl.pallas_call(body, out_shape=out_shape)(...)


# baseline (device time: 238253 ns/iter reference)
import jax
import jax.numpy as jnp
from jax import lax
from jax.experimental import pallas as pl
from jax.experimental.pallas import tpu as pltpu

M = 8192
N = 2048
H = N // 2


def kernel(x):
    xb = x.reshape(M, N).astype(jnp.bfloat16)

    def body(x_ref, out_ref, recv_buf, send_sem, recv_sem, copy_sem):
        my_x = lax.axis_index("x")
        my_y = lax.axis_index("y")
        peer_y = 1 - my_y

        barrier_sem = pltpu.get_barrier_semaphore()
        pl.semaphore_signal(
            barrier_sem,
            inc=1,
            device_id=(my_x, peer_y),
            device_id_type=pl.DeviceIdType.MESH,
        )
        pl.semaphore_wait(barrier_sem, 1)

        def exchange(send_start, keep_start):
            rdma = pltpu.make_async_remote_copy(
                src_ref=x_ref.at[:, pl.ds(send_start, H)],
                dst_ref=recv_buf,
                send_sem=send_sem,
                recv_sem=recv_sem,
                device_id=(my_x, peer_y),
                device_id_type=pl.DeviceIdType.MESH,
            )
            rdma.start()
            local = pltpu.make_async_copy(
                x_ref.at[:, pl.ds(keep_start, H)], out_ref, copy_sem
            )
            local.start()
            local.wait()
            rdma.wait()
            out_ref[...] = out_ref[...] + recv_buf[...]

        @pl.when(my_y == 0)
        def _():
            exchange(H, 0)

        @pl.when(my_y == 1)
        def _():
            exchange(0, H)

    return pl.pallas_call(
        body,
        out_shape=jax.ShapeDtypeStruct((M, H), jnp.bfloat16),
        in_specs=[pl.BlockSpec(memory_space=pltpu.MemorySpace.HBM)],
        out_specs=pl.BlockSpec(memory_space=pltpu.VMEM),
        scratch_shapes=[
            pltpu.VMEM((M, H), jnp.bfloat16),
            pltpu.SemaphoreType.DMA,
            pltpu.SemaphoreType.DMA,
            pltpu.SemaphoreType.DMA,
        ],
        compiler_params=pltpu.CompilerParams(
            collective_id=0, vmem_limit_bytes=60 * 1024 * 1024
        ),
    )(xb)


# device time: 155537 ns/iter; 1.5318x vs baseline; 1.5318x over previous
import jax
import jax.numpy as jnp
from jax import lax
from jax.experimental import pallas as pl
from jax.experimental.pallas import tpu as pltpu

M = 8192
N = 2048
H = N // 2
MH = M // 2
NC = 16
RC = MH // NC


def kernel(x):
    xb = x.reshape(M, N).astype(jnp.bfloat16)

    def body(
        x_ref,
        out_ref,
        recv_buf,
        send_y,
        recv_y,
        send_x,
        recv_x,
        copy_sem,
    ):
        my_x = lax.axis_index("x")
        my_y = lax.axis_index("y")
        peer_y = 1 - my_y
        peer_x = 1 - my_x

        send_col = peer_y * H
        keep_col = my_y * H
        my_row0 = my_x * MH
        ot_row0 = peer_x * MH

        barrier_sem = pltpu.get_barrier_semaphore()
        for dev in ((my_x, peer_y), (peer_x, my_y)):
            pl.semaphore_signal(
                barrier_sem,
                inc=1,
                device_id=dev,
                device_id_type=pl.DeviceIdType.MESH,
            )
        pl.semaphore_wait(barrier_sem, 2)

        local = pltpu.make_async_copy(
            x_ref.at[:, pl.ds(keep_col, H)], out_ref, copy_sem
        )
        local.start()

        y_rdmas = []
        for c in range(NC):
            r0 = my_row0 + c * RC
            rd = pltpu.make_async_remote_copy(
                src_ref=x_ref.at[pl.ds(r0, RC), pl.ds(send_col, H)],
                dst_ref=recv_buf.at[pl.ds(r0, RC), :],
                send_sem=send_y.at[c],
                recv_sem=recv_y.at[c],
                device_id=(my_x, peer_y),
                device_id_type=pl.DeviceIdType.MESH,
            )
            rd.start()
            y_rdmas.append(rd)

        local.wait()

        x_rdmas = []
        for c in range(NC):
            r0 = my_row0 + c * RC
            y_rdmas[c].wait_recv()
            rd = pltpu.make_async_remote_copy(
                src_ref=recv_buf.at[pl.ds(r0, RC), :],
                dst_ref=recv_buf.at[pl.ds(r0, RC), :],
                send_sem=send_x.at[c],
                recv_sem=recv_x.at[c],
                device_id=(peer_x, my_y),
                device_id_type=pl.DeviceIdType.MESH,
            )
            rd.start()
            x_rdmas.append(rd)
            out_ref[pl.ds(r0, RC), :] = (
                out_ref[pl.ds(r0, RC), :] + recv_buf[pl.ds(r0, RC), :]
            )

        for c in range(NC):
            r0 = ot_row0 + c * RC
            x_rdmas[c].wait_recv()
            out_ref[pl.ds(r0, RC), :] = (
                out_ref[pl.ds(r0, RC), :] + recv_buf[pl.ds(r0, RC), :]
            )

        for c in range(NC):
            y_rdmas[c].wait_send()
            x_rdmas[c].wait_send()

    return pl.pallas_call(
        body,
        out_shape=jax.ShapeDtypeStruct((M, H), jnp.bfloat16),
        in_specs=[pl.BlockSpec(memory_space=pltpu.MemorySpace.HBM)],
        out_specs=pl.BlockSpec(memory_space=pltpu.VMEM),
        scratch_shapes=[
            pltpu.VMEM((M, H), jnp.bfloat16),
            pltpu.SemaphoreType.DMA((NC,)),
            pltpu.SemaphoreType.DMA((NC,)),
            pltpu.SemaphoreType.DMA((NC,)),
            pltpu.SemaphoreType.DMA((NC,)),
            pltpu.SemaphoreType.DMA,
        ],
        compiler_params=pltpu.CompilerParams(
            collective_id=0, vmem_limit_bytes=60 * 1024 * 1024
        ),
    )(xb)
